# baseline (speedup 1.0000x reference)
import functools

import jax
import jax.numpy as jnp
from jax import lax
from jax.experimental import pallas as pl
from jax.experimental.pallas import tpu as pltpu
from jax.experimental.pallas import tpu_sc as plsc

N_TOK = 2048
D_MODEL = 1024
D_OUT = 1024
N_EXPERTS = 8

B = 128
NB = N_TOK // B + N_EXPERTS
NP = NB * B

_SC_INFO = plsc.get_sparse_core_info()
_NC, _NS = _SC_INFO.num_cores, _SC_INFO.num_subcores
_NW = _NC * _NS


def _make_sc_row_gather(n_rows, d):
    assert n_rows % (8 * _NW) == 0 and d % 16 == 0
    rows_per_w = n_rows // _NW
    mesh = plsc.VectorSubcoreMesh(core_axis_name="c", subcore_axis_name="s")

    @functools.partial(
        pl.kernel, mesh=mesh,
        out_type=jax.ShapeDtypeStruct((n_rows, d), jnp.float32),
        scratch_types=[
            pltpu.VMEM((rows_per_w,), jnp.int32),
            pltpu.VMEM((rows_per_w, d), jnp.float32),
            pltpu.SemaphoreType.DMA,
        ],
    )
    def gather(table_hbm, idx_hbm, out_hbm, idx_v, rows_v, sem):
        wid = lax.axis_index("s") * _NC + lax.axis_index("c")
        base = wid * rows_per_w
        pltpu.sync_copy(idx_hbm.at[pl.ds(base, rows_per_w)], idx_v)
        pltpu.async_copy(table_hbm.at[idx_v], rows_v, sem).wait()
        pltpu.sync_copy(rows_v, out_hbm.at[pl.ds(base, rows_per_w)])

    return gather


_sc_gather_dispatch = _make_sc_row_gather(NP, D_MODEL)
_sc_gather_combine = _make_sc_row_gather(N_TOK, D_OUT)


def _gemm_block(meta_ref, xs_ref, w1_ref, b1_ref, w2_ref, b2_ref, y_ref):
    g = pl.program_id(0)

    @pl.when(meta_ref[NB + g] == 1)
    def _():
        xb = xs_ref[...]
        h = jnp.dot(xb, w1_ref[0], preferred_element_type=jnp.float32,
                    precision=lax.Precision.DEFAULT)
        h = h + b1_ref[0]
        y = jnp.dot(h, w2_ref[0], preferred_element_type=jnp.float32,
                    precision=lax.Precision.DEFAULT)
        y_ref[...] = y + b2_ref[0]


@jax.jit
def kernel(x, moe_weight, W1, b1, W2, b2):
    idx = jnp.argmax(moe_weight, axis=1).astype(jnp.int32)
    gate = jnp.max(moe_weight, axis=1)

    oh = jax.nn.one_hot(idx, N_EXPERTS, dtype=jnp.int32)
    counts = jnp.sum(oh, axis=0)
    rank = jnp.take_along_axis(jnp.cumsum(oh, axis=0), idx[:, None], 1)[:, 0] - 1
    blk_per_e = (counts + B - 1) // B
    blk_start = jnp.concatenate([jnp.zeros((1,), jnp.int32),
                                 jnp.cumsum(blk_per_e)[:-1].astype(jnp.int32)])
    total_blocks = jnp.sum(blk_per_e)
    pad_start = blk_start * B
    slot = pad_start[idx] + rank

    perm = (jnp.arange(NP, dtype=jnp.int32) % N_TOK).at[slot].set(
        jnp.arange(N_TOK, dtype=jnp.int32), unique_indices=True,
        mode="promise_in_bounds")

    gblk = jnp.arange(NB, dtype=jnp.int32)
    block_expert = (jnp.searchsorted(blk_start, gblk, side="right") - 1
                    ).astype(jnp.int32)
    block_expert = jnp.clip(block_expert, 0, N_EXPERTS - 1)
    valid = (gblk < total_blocks).astype(jnp.int32)
    meta = jnp.concatenate([block_expert, valid])

    xs = _sc_gather_dispatch(x, perm)

    grid_spec = pltpu.PrefetchScalarGridSpec(
        num_scalar_prefetch=1,
        grid=(NB,),
        in_specs=[
            pl.BlockSpec((B, D_MODEL), lambda g, m: (g, 0)),
            pl.BlockSpec((1, D_MODEL, D_MODEL), lambda g, m: (m[g], 0, 0)),
            pl.BlockSpec((1, 1, D_MODEL), lambda g, m: (m[g], 0, 0)),
            pl.BlockSpec((1, D_MODEL, D_OUT), lambda g, m: (m[g], 0, 0)),
            pl.BlockSpec((1, 1, D_OUT), lambda g, m: (m[g], 0, 0)),
        ],
        out_specs=pl.BlockSpec((B, D_OUT), lambda g, m: (g, 0)),
    )
    y = pl.pallas_call(
        _gemm_block,
        grid_spec=grid_spec,
        out_shape=jax.ShapeDtypeStruct((NP, D_OUT), jnp.float32),
    )(meta, xs, W1, b1.reshape(N_EXPERTS, 1, D_MODEL), W2,
      b2.reshape(N_EXPERTS, 1, D_OUT))

    return _sc_gather_combine(y, slot) * gate[:, None]


# speedup vs baseline: 1.0371x; 1.0371x over previous
import functools

import jax
import jax.numpy as jnp
from jax import lax
from jax.experimental import pallas as pl
from jax.experimental.pallas import tpu as pltpu
from jax.experimental.pallas import tpu_sc as plsc

N_TOK = 2048
D_MODEL = 1024
D_OUT = 1024
N_EXPERTS = 8

B = 128
NB = N_TOK // B + N_EXPERTS
NP = NB * B

_SC_INFO = plsc.get_sparse_core_info()
_NC, _NS = _SC_INFO.num_cores, _SC_INFO.num_subcores
_NW = _NC * _NS


def _make_sc_row_gather(n_rows, d):
    assert n_rows % (8 * _NW) == 0 and d % 16 == 0
    rows_per_w = n_rows // _NW
    mesh = plsc.VectorSubcoreMesh(core_axis_name="c", subcore_axis_name="s")

    @functools.partial(
        pl.kernel, mesh=mesh,
        out_type=jax.ShapeDtypeStruct((n_rows, d), jnp.float32),
        scratch_types=[
            pltpu.VMEM((rows_per_w,), jnp.int32),
            pltpu.VMEM((rows_per_w, d), jnp.float32),
            pltpu.SemaphoreType.DMA,
        ],
    )
    def gather(table_hbm, idx_hbm, out_hbm, idx_v, rows_v, sem):
        wid = lax.axis_index("s") * _NC + lax.axis_index("c")
        base = wid * rows_per_w
        pltpu.sync_copy(idx_hbm.at[pl.ds(base, rows_per_w)], idx_v)
        pltpu.async_copy(table_hbm.at[idx_v], rows_v, sem).wait()
        pltpu.sync_copy(rows_v, out_hbm.at[pl.ds(base, rows_per_w)])

    return gather


_sc_gather_dispatch = _make_sc_row_gather(NP, D_MODEL)
_sc_gather_combine = _make_sc_row_gather(N_TOK, D_OUT)


def _gemm_block(meta_ref, xs_ref, w1_ref, b1_ref, w2_ref, b2_ref, gate_ref,
                y_ref):
    g = pl.program_id(0)

    @pl.when(meta_ref[NB + g] == 1)
    def _():
        xb = xs_ref[...]
        h = jnp.dot(xb, w1_ref[0], preferred_element_type=jnp.float32,
                    precision=lax.Precision.DEFAULT)
        h = h + b1_ref[0]
        y = jnp.dot(h, w2_ref[0], preferred_element_type=jnp.float32,
                    precision=lax.Precision.DEFAULT)
        y = y + b2_ref[0]
        y_ref[...] = y * gate_ref[0, 0, :][:, None]


@jax.jit
def kernel(x, moe_weight, W1, b1, W2, b2):
    idx = jnp.argmax(moe_weight, axis=1).astype(jnp.int32)
    gate = jnp.max(moe_weight, axis=1)

    oh = jax.nn.one_hot(idx, N_EXPERTS, dtype=jnp.int32)
    counts = jnp.sum(oh, axis=0)
    rank = jnp.take_along_axis(jnp.cumsum(oh, axis=0), idx[:, None], 1)[:, 0] - 1
    blk_per_e = (counts + B - 1) // B
    blk_start = jnp.concatenate([jnp.zeros((1,), jnp.int32),
                                 jnp.cumsum(blk_per_e)[:-1].astype(jnp.int32)])
    total_blocks = jnp.sum(blk_per_e)
    pad_start = blk_start * B
    slot = pad_start[idx] + rank

    perm = (jnp.arange(NP, dtype=jnp.int32) % N_TOK).at[slot].set(
        jnp.arange(N_TOK, dtype=jnp.int32), unique_indices=True,
        mode="promise_in_bounds")
    gate_sorted = jnp.zeros((NP,), jnp.float32).at[slot].set(
        gate, unique_indices=True, mode="promise_in_bounds")

    gblk = jnp.arange(NB, dtype=jnp.int32)
    block_expert = (jnp.searchsorted(blk_start, gblk, side="right") - 1
                    ).astype(jnp.int32)
    block_expert = jnp.clip(block_expert, 0, N_EXPERTS - 1)
    valid = (gblk < total_blocks).astype(jnp.int32)
    meta = jnp.concatenate([block_expert, valid])

    xs = _sc_gather_dispatch(x, perm)

    grid_spec = pltpu.PrefetchScalarGridSpec(
        num_scalar_prefetch=1,
        grid=(NB,),
        in_specs=[
            pl.BlockSpec((B, D_MODEL), lambda g, m: (g, 0)),
            pl.BlockSpec((1, D_MODEL, D_MODEL), lambda g, m: (m[g], 0, 0)),
            pl.BlockSpec((1, 1, D_MODEL), lambda g, m: (m[g], 0, 0)),
            pl.BlockSpec((1, D_MODEL, D_OUT), lambda g, m: (m[g], 0, 0)),
            pl.BlockSpec((1, 1, D_OUT), lambda g, m: (m[g], 0, 0)),
            pl.BlockSpec((1, 1, B), lambda g, m: (g, 0, 0)),
        ],
        out_specs=pl.BlockSpec((B, D_OUT), lambda g, m: (g, 0)),
    )
    y = pl.pallas_call(
        _gemm_block,
        grid_spec=grid_spec,
        out_shape=jax.ShapeDtypeStruct((NP, D_OUT), jnp.float32),
    )(meta, xs, W1, b1.reshape(N_EXPERTS, 1, D_MODEL), W2,
      b2.reshape(N_EXPERTS, 1, D_OUT), gate_sorted.reshape(NB, 1, B))

    return _sc_gather_combine(y, slot)


# speedup vs baseline: 1.1582x; 1.1168x over previous
import functools

import jax
import jax.numpy as jnp
from jax import lax
from jax.experimental import pallas as pl
from jax.experimental.pallas import tpu as pltpu
from jax.experimental.pallas import tpu_sc as plsc

N_TOK = 2048
D_MODEL = 1024
D_OUT = 1024
N_EXPERTS = 8

B = 128
NB = N_TOK // B + N_EXPERTS
NP = NB * B

_NC, _NS = 2, 16
_NW = _NC * _NS


def _make_sc_row_gather(n_rows, d):
    assert n_rows % (8 * _NW) == 0 and d % 16 == 0
    rows_per_w = n_rows // _NW
    mesh = plsc.VectorSubcoreMesh(core_axis_name="c", subcore_axis_name="s")

    @functools.partial(
        pl.kernel, mesh=mesh,
        out_type=jax.ShapeDtypeStruct((n_rows, d), jnp.float32),
        scratch_types=[
            pltpu.VMEM((rows_per_w,), jnp.int32),
            pltpu.VMEM((rows_per_w, d), jnp.float32),
            pltpu.SemaphoreType.DMA,
        ],
    )
    def gather(table_hbm, idx_hbm, out_hbm, idx_v, rows_v, sem):
        wid = lax.axis_index("s") * _NC + lax.axis_index("c")
        base = wid * rows_per_w
        pltpu.sync_copy(idx_hbm.at[pl.ds(base, rows_per_w)], idx_v)
        pltpu.async_copy(table_hbm.at[idx_v], rows_v, sem).wait()
        pltpu.sync_copy(rows_v, out_hbm.at[pl.ds(base, rows_per_w)])

    return gather


_sc_gather_dispatch = _make_sc_row_gather(NP, D_MODEL)
_sc_gather_combine = _make_sc_row_gather(N_TOK, D_OUT)


def _gemm_block(meta_ref, xs_ref, w1_ref, b1_ref, w2_ref, b2_ref, gate_ref,
                y_ref):
    g = pl.program_id(0)

    @pl.when(meta_ref[NB + g] == 1)
    def _():
        xb = xs_ref[...]
        h = jnp.dot(xb, w1_ref[0], preferred_element_type=jnp.float32,
                    precision=lax.Precision.DEFAULT)
        h = h + b1_ref[0]
        y = jnp.dot(h, w2_ref[0], preferred_element_type=jnp.float32,
                    precision=lax.Precision.DEFAULT)
        y = y + b2_ref[0]
        y_ref[...] = y * gate_ref[0, 0, :][:, None]


@jax.jit
def kernel(x, moe_weight, W1, b1, W2, b2):
    idx = jnp.argmax(moe_weight, axis=1).astype(jnp.int32)
    gate = jnp.max(moe_weight, axis=1)

    ohb = idx[:, None] == jnp.arange(N_EXPERTS, dtype=jnp.int32)[None, :]
    oh = ohb.astype(jnp.int32)
    csum = jnp.cumsum(oh, axis=0)
    counts = csum[-1]
    rank = jnp.sum(csum * oh, axis=1) - 1
    blk_per_e = (counts + B - 1) // B
    blk_start = jnp.concatenate([jnp.zeros((1,), jnp.int32),
                                 jnp.cumsum(blk_per_e)[:-1].astype(jnp.int32)])
    total_blocks = jnp.sum(blk_per_e)
    pad_start = blk_start * B
    slot = jnp.sum(jnp.where(ohb, pad_start[None, :], 0), axis=1) + rank

    perm = (jnp.arange(NP, dtype=jnp.int32) % N_TOK).at[slot].set(
        jnp.arange(N_TOK, dtype=jnp.int32), unique_indices=True,
        mode="promise_in_bounds")
    gate_sorted = jnp.zeros((NP,), jnp.float32).at[slot].set(
        gate, unique_indices=True, mode="promise_in_bounds")

    gblk = jnp.arange(NB, dtype=jnp.int32)
    block_expert = jnp.sum(
        (gblk[:, None] >= blk_start[None, :]).astype(jnp.int32), axis=1) - 1
    valid = (gblk < total_blocks).astype(jnp.int32)
    meta = jnp.concatenate([block_expert, valid])

    xs = _sc_gather_dispatch(x, perm)

    grid_spec = pltpu.PrefetchScalarGridSpec(
        num_scalar_prefetch=1,
        grid=(NB,),
        in_specs=[
            pl.BlockSpec((B, D_MODEL), lambda g, m: (g, 0)),
            pl.BlockSpec((1, D_MODEL, D_MODEL), lambda g, m: (m[g], 0, 0)),
            pl.BlockSpec((1, 1, D_MODEL), lambda g, m: (m[g], 0, 0)),
            pl.BlockSpec((1, D_MODEL, D_OUT), lambda g, m: (m[g], 0, 0)),
            pl.BlockSpec((1, 1, D_OUT), lambda g, m: (m[g], 0, 0)),
            pl.BlockSpec((1, 1, B), lambda g, m: (g, 0, 0)),
        ],
        out_specs=pl.BlockSpec((B, D_OUT), lambda g, m: (g, 0)),
    )
    y = pl.pallas_call(
        _gemm_block,
        grid_spec=grid_spec,
        out_shape=jax.ShapeDtypeStruct((NP, D_OUT), jnp.float32),
    )(meta, xs, W1, b1.reshape(N_EXPERTS, 1, D_MODEL), W2,
      b2.reshape(N_EXPERTS, 1, D_OUT), gate_sorted.reshape(NB, 1, B))

    return _sc_gather_combine(y, slot)


# speedup vs baseline: 1.2588x; 1.0868x over previous
import functools

import jax
import jax.numpy as jnp
from jax import lax
from jax.experimental import pallas as pl
from jax.experimental.pallas import tpu as pltpu
from jax.experimental.pallas import tpu_sc as plsc

N_TOK = 2048
D_MODEL = 1024
D_OUT = 1024
N_EXPERTS = 8

B = 128
NB = N_TOK // B + N_EXPERTS
NP = NB * B

_NC, _NS = 2, 16
_NW = _NC * _NS


def _make_sc_row_gather(n_rows, d):
    assert n_rows % (8 * _NW) == 0 and d % 16 == 0
    rows_per_w = n_rows // _NW
    mesh = plsc.VectorSubcoreMesh(core_axis_name="c", subcore_axis_name="s")

    @functools.partial(
        pl.kernel, mesh=mesh,
        out_type=jax.ShapeDtypeStruct((n_rows, d), jnp.float32),
        scratch_types=[
            pltpu.VMEM((rows_per_w,), jnp.int32),
            pltpu.VMEM((rows_per_w, d), jnp.float32),
            pltpu.SemaphoreType.DMA,
        ],
    )
    def gather(table_hbm, idx_hbm, out_hbm, idx_v, rows_v, sem):
        wid = lax.axis_index("s") * _NC + lax.axis_index("c")
        base = wid * rows_per_w
        pltpu.sync_copy(idx_hbm.at[pl.ds(base, rows_per_w)], idx_v)
        pltpu.async_copy(table_hbm.at[idx_v], rows_v, sem).wait()
        pltpu.sync_copy(rows_v, out_hbm.at[pl.ds(base, rows_per_w)])

    return gather


def _make_sc_row_scatter(n_src, n_dst, d):
    assert n_src % (8 * _NW) == 0 and d % 16 == 0
    rows_per_w = n_src // _NW
    mesh = plsc.VectorSubcoreMesh(core_axis_name="c", subcore_axis_name="s")

    @functools.partial(
        pl.kernel, mesh=mesh,
        out_type=jax.ShapeDtypeStruct((n_dst, d), jnp.float32),
        scratch_types=[
            pltpu.VMEM((rows_per_w,), jnp.int32),
            pltpu.VMEM((rows_per_w, d), jnp.float32),
            pltpu.SemaphoreType.DMA,
        ],
    )
    def scatter(x_hbm, idx_hbm, out_hbm, idx_v, rows_v, sem):
        wid = lax.axis_index("s") * _NC + lax.axis_index("c")
        base = wid * rows_per_w
        pltpu.sync_copy(idx_hbm.at[pl.ds(base, rows_per_w)], idx_v)
        pltpu.sync_copy(x_hbm.at[pl.ds(base, rows_per_w)], rows_v)
        pltpu.async_copy(rows_v, out_hbm.at[idx_v], sem).wait()

    return scatter


_sc_scatter_dispatch = _make_sc_row_scatter(N_TOK, NP, D_MODEL)
_sc_gather_combine = _make_sc_row_gather(N_TOK, D_OUT)


def _gemm_block(meta_ref, xs_ref, w1_ref, b1_ref, w2_ref, b2_ref, gate_ref,
                y_ref):
    g = pl.program_id(0)

    @pl.when(meta_ref[NB + g] == 1)
    def _():
        xb = xs_ref[...]
        h = jnp.dot(xb, w1_ref[0], preferred_element_type=jnp.float32,
                    precision=lax.Precision.DEFAULT)
        h = h + b1_ref[0]
        y = jnp.dot(h, w2_ref[0], preferred_element_type=jnp.float32,
                    precision=lax.Precision.DEFAULT)
        y = y + b2_ref[0]
        y_ref[...] = y * gate_ref[0, 0, :][:, None]


@jax.jit
def kernel(x, moe_weight, W1, b1, W2, b2):
    idx = jnp.argmax(moe_weight, axis=1).astype(jnp.int32)
    gate = jnp.max(moe_weight, axis=1)

    ohb = idx[:, None] == jnp.arange(N_EXPERTS, dtype=jnp.int32)[None, :]
    oh = ohb.astype(jnp.int32)
    csum = jnp.cumsum(oh, axis=0)
    counts = csum[-1]
    rank = jnp.sum(csum * oh, axis=1) - 1
    blk_per_e = (counts + B - 1) // B
    blk_start = jnp.concatenate([jnp.zeros((1,), jnp.int32),
                                 jnp.cumsum(blk_per_e)[:-1].astype(jnp.int32)])
    total_blocks = jnp.sum(blk_per_e)
    pad_start = blk_start * B
    slot = jnp.sum(jnp.where(ohb, pad_start[None, :], 0), axis=1) + rank

    gate_sorted = jnp.zeros((NP,), jnp.float32).at[slot].set(
        gate, unique_indices=True, mode="promise_in_bounds")

    gblk = jnp.arange(NB, dtype=jnp.int32)
    block_expert = jnp.sum(
        (gblk[:, None] >= blk_start[None, :]).astype(jnp.int32), axis=1) - 1
    valid = (gblk < total_blocks).astype(jnp.int32)
    meta = jnp.concatenate([block_expert, valid])

    xs = _sc_scatter_dispatch(x, slot)

    grid_spec = pltpu.PrefetchScalarGridSpec(
        num_scalar_prefetch=1,
        grid=(NB,),
        in_specs=[
            pl.BlockSpec((B, D_MODEL), lambda g, m: (g, 0)),
            pl.BlockSpec((1, D_MODEL, D_MODEL), lambda g, m: (m[g], 0, 0)),
            pl.BlockSpec((1, 1, D_MODEL), lambda g, m: (m[g], 0, 0)),
            pl.BlockSpec((1, D_MODEL, D_OUT), lambda g, m: (m[g], 0, 0)),
            pl.BlockSpec((1, 1, D_OUT), lambda g, m: (m[g], 0, 0)),
            pl.BlockSpec((1, 1, B), lambda g, m: (g, 0, 0)),
        ],
        out_specs=pl.BlockSpec((B, D_OUT), lambda g, m: (g, 0)),
    )
    y = pl.pallas_call(
        _gemm_block,
        grid_spec=grid_spec,
        out_shape=jax.ShapeDtypeStruct((NP, D_OUT), jnp.float32),
    )(meta, xs, W1, b1.reshape(N_EXPERTS, 1, D_MODEL), W2,
      b2.reshape(N_EXPERTS, 1, D_OUT), gate_sorted.reshape(NB, 1, B))

    return _sc_gather_combine(y, slot)


# speedup vs baseline: 1.3757x; 1.0929x over previous
import functools

import jax
import jax.numpy as jnp
from jax import lax
from jax.experimental import pallas as pl
from jax.experimental.pallas import tpu as pltpu
from jax.experimental.pallas import tpu_sc as plsc

N_TOK = 2048
D_MODEL = 1024
D_OUT = 1024
N_EXPERTS = 8

B = 256
NB = N_TOK // B + N_EXPERTS
NP = NB * B

_NC, _NS = 2, 16
_NW = _NC * _NS


def _make_sc_row_gather(n_rows, d):
    assert n_rows % (8 * _NW) == 0 and d % 16 == 0
    rows_per_w = n_rows // _NW
    mesh = plsc.VectorSubcoreMesh(core_axis_name="c", subcore_axis_name="s")

    @functools.partial(
        pl.kernel, mesh=mesh,
        out_type=jax.ShapeDtypeStruct((n_rows, d), jnp.float32),
        scratch_types=[
            pltpu.VMEM((rows_per_w,), jnp.int32),
            pltpu.VMEM((rows_per_w, d), jnp.float32),
            pltpu.SemaphoreType.DMA,
        ],
    )
    def gather(table_hbm, idx_hbm, out_hbm, idx_v, rows_v, sem):
        wid = lax.axis_index("s") * _NC + lax.axis_index("c")
        base = wid * rows_per_w
        pltpu.sync_copy(idx_hbm.at[pl.ds(base, rows_per_w)], idx_v)
        pltpu.async_copy(table_hbm.at[idx_v], rows_v, sem).wait()
        pltpu.sync_copy(rows_v, out_hbm.at[pl.ds(base, rows_per_w)])

    return gather


def _make_sc_row_scatter(n_src, n_dst, d):
    assert n_src % (8 * _NW) == 0 and d % 16 == 0
    rows_per_w = n_src // _NW
    mesh = plsc.VectorSubcoreMesh(core_axis_name="c", subcore_axis_name="s")

    @functools.partial(
        pl.kernel, mesh=mesh,
        out_type=jax.ShapeDtypeStruct((n_dst, d), jnp.float32),
        scratch_types=[
            pltpu.VMEM((rows_per_w,), jnp.int32),
            pltpu.VMEM((rows_per_w, d), jnp.float32),
            pltpu.SemaphoreType.DMA,
        ],
    )
    def scatter(x_hbm, idx_hbm, out_hbm, idx_v, rows_v, sem):
        wid = lax.axis_index("s") * _NC + lax.axis_index("c")
        base = wid * rows_per_w
        pltpu.sync_copy(idx_hbm.at[pl.ds(base, rows_per_w)], idx_v)
        pltpu.sync_copy(x_hbm.at[pl.ds(base, rows_per_w)], rows_v)
        pltpu.async_copy(rows_v, out_hbm.at[idx_v], sem).wait()

    return scatter


_sc_scatter_dispatch = _make_sc_row_scatter(N_TOK, NP, D_MODEL)
_sc_gather_combine = _make_sc_row_gather(N_TOK, D_OUT)


def _gemm_block(meta_ref, xs_ref, w1_ref, b1_ref, w2_ref, b2_ref, gate_ref,
                y_ref):
    g = pl.program_id(0)

    @pl.when(meta_ref[NB + g] == 1)
    def _():
        xb = xs_ref[...]
        h = jnp.dot(xb, w1_ref[0], preferred_element_type=jnp.float32,
                    precision=lax.Precision.DEFAULT)
        h = h + b1_ref[0]
        y = jnp.dot(h, w2_ref[0], preferred_element_type=jnp.float32,
                    precision=lax.Precision.DEFAULT)
        y = y + b2_ref[0]
        y_ref[...] = y * gate_ref[0, 0, :][:, None]


@jax.jit
def kernel(x, moe_weight, W1, b1, W2, b2):
    idx = jnp.argmax(moe_weight, axis=1).astype(jnp.int32)
    gate = jnp.max(moe_weight, axis=1)

    ohb = idx[:, None] == jnp.arange(N_EXPERTS, dtype=jnp.int32)[None, :]
    oh = ohb.astype(jnp.int32)
    csum = jnp.cumsum(oh, axis=0)
    counts = csum[-1]
    rank = jnp.sum(csum * oh, axis=1) - 1
    blk_per_e = (counts + B - 1) // B
    blk_start = jnp.concatenate([jnp.zeros((1,), jnp.int32),
                                 jnp.cumsum(blk_per_e)[:-1].astype(jnp.int32)])
    total_blocks = jnp.sum(blk_per_e)
    pad_start = blk_start * B
    slot = jnp.sum(jnp.where(ohb, pad_start[None, :], 0), axis=1) + rank

    gate_sorted = jnp.zeros((NP,), jnp.float32).at[slot].set(
        gate, unique_indices=True, mode="promise_in_bounds")

    gblk = jnp.arange(NB, dtype=jnp.int32)
    block_expert = jnp.sum(
        (gblk[:, None] >= blk_start[None, :]).astype(jnp.int32), axis=1) - 1
    valid = (gblk < total_blocks).astype(jnp.int32)
    meta = jnp.concatenate([block_expert, valid])

    xs = _sc_scatter_dispatch(x, slot)

    grid_spec = pltpu.PrefetchScalarGridSpec(
        num_scalar_prefetch=1,
        grid=(NB,),
        in_specs=[
            pl.BlockSpec((B, D_MODEL), lambda g, m: (g, 0)),
            pl.BlockSpec((1, D_MODEL, D_MODEL), lambda g, m: (m[g], 0, 0)),
            pl.BlockSpec((1, 1, D_MODEL), lambda g, m: (m[g], 0, 0)),
            pl.BlockSpec((1, D_MODEL, D_OUT), lambda g, m: (m[g], 0, 0)),
            pl.BlockSpec((1, 1, D_OUT), lambda g, m: (m[g], 0, 0)),
            pl.BlockSpec((1, 1, B), lambda g, m: (g, 0, 0)),
        ],
        out_specs=pl.BlockSpec((B, D_OUT), lambda g, m: (g, 0)),
    )
    y = pl.pallas_call(
        _gemm_block,
        grid_spec=grid_spec,
        out_shape=jax.ShapeDtypeStruct((NP, D_OUT), jnp.float32),
    )(meta, xs, W1, b1.reshape(N_EXPERTS, 1, D_MODEL), W2,
      b2.reshape(N_EXPERTS, 1, D_OUT), gate_sorted.reshape(NB, 1, B))

    return _sc_gather_combine(y, slot)


# speedup vs baseline: 1.4526x; 1.0559x over previous
import functools

import jax
import jax.numpy as jnp
from jax import lax
from jax.experimental import pallas as pl
from jax.experimental.pallas import tpu as pltpu
from jax.experimental.pallas import tpu_sc as plsc

N_TOK = 2048
D_MODEL = 1024
D_OUT = 1024
N_EXPERTS = 8

B = 320
NB = N_TOK // B + N_EXPERTS
NP = NB * B

_NC, _NS = 2, 16
_NW = _NC * _NS


def _make_sc_row_gather(n_rows, d):
    assert n_rows % (8 * _NW) == 0 and d % 16 == 0
    rows_per_w = n_rows // _NW
    mesh = plsc.VectorSubcoreMesh(core_axis_name="c", subcore_axis_name="s")

    @functools.partial(
        pl.kernel, mesh=mesh,
        out_type=jax.ShapeDtypeStruct((n_rows, d), jnp.float32),
        scratch_types=[
            pltpu.VMEM((rows_per_w,), jnp.int32),
            pltpu.VMEM((rows_per_w, d), jnp.float32),
            pltpu.SemaphoreType.DMA,
        ],
    )
    def gather(table_hbm, idx_hbm, out_hbm, idx_v, rows_v, sem):
        wid = lax.axis_index("s") * _NC + lax.axis_index("c")
        base = wid * rows_per_w
        pltpu.sync_copy(idx_hbm.at[pl.ds(base, rows_per_w)], idx_v)
        pltpu.async_copy(table_hbm.at[idx_v], rows_v, sem).wait()
        pltpu.sync_copy(rows_v, out_hbm.at[pl.ds(base, rows_per_w)])

    return gather


def _make_sc_row_scatter(n_src, n_dst, d):
    assert n_src % (8 * _NW) == 0 and d % 16 == 0
    rows_per_w = n_src // _NW
    mesh = plsc.VectorSubcoreMesh(core_axis_name="c", subcore_axis_name="s")

    @functools.partial(
        pl.kernel, mesh=mesh,
        out_type=jax.ShapeDtypeStruct((n_dst, d), jnp.float32),
        scratch_types=[
            pltpu.VMEM((rows_per_w,), jnp.int32),
            pltpu.VMEM((rows_per_w, d), jnp.float32),
            pltpu.SemaphoreType.DMA,
        ],
    )
    def scatter(x_hbm, idx_hbm, out_hbm, idx_v, rows_v, sem):
        wid = lax.axis_index("s") * _NC + lax.axis_index("c")
        base = wid * rows_per_w
        pltpu.sync_copy(idx_hbm.at[pl.ds(base, rows_per_w)], idx_v)
        pltpu.sync_copy(x_hbm.at[pl.ds(base, rows_per_w)], rows_v)
        pltpu.async_copy(rows_v, out_hbm.at[idx_v], sem).wait()

    return scatter


_sc_scatter_dispatch = _make_sc_row_scatter(N_TOK, NP, D_MODEL)
_sc_gather_combine = _make_sc_row_gather(N_TOK, D_OUT)


def _gemm_block(meta_ref, xs_ref, w1_ref, b1_ref, w2_ref, b2_ref, gate_ref,
                y_ref):
    g = pl.program_id(0)

    @pl.when(meta_ref[NB + g] == 1)
    def _():
        xb = xs_ref[...]
        h = jnp.dot(xb, w1_ref[0], preferred_element_type=jnp.float32,
                    precision=lax.Precision.DEFAULT)
        h = h + b1_ref[0]
        y = jnp.dot(h, w2_ref[0], preferred_element_type=jnp.float32,
                    precision=lax.Precision.DEFAULT)
        y = y + b2_ref[0]
        y_ref[...] = y * gate_ref[0, 0, :][:, None]


@jax.jit
def kernel(x, moe_weight, W1, b1, W2, b2):
    idx = jnp.argmax(moe_weight, axis=1).astype(jnp.int32)
    gate = jnp.max(moe_weight, axis=1)

    ohb = idx[:, None] == jnp.arange(N_EXPERTS, dtype=jnp.int32)[None, :]
    oh = ohb.astype(jnp.int32)
    csum = jnp.cumsum(oh, axis=0)
    counts = csum[-1]
    rank = jnp.sum(csum * oh, axis=1) - 1
    blk_per_e = (counts + B - 1) // B
    blk_start = jnp.concatenate([jnp.zeros((1,), jnp.int32),
                                 jnp.cumsum(blk_per_e)[:-1].astype(jnp.int32)])
    total_blocks = jnp.sum(blk_per_e)
    pad_start = blk_start * B
    slot = jnp.sum(jnp.where(ohb, pad_start[None, :], 0), axis=1) + rank

    gate_sorted = jnp.zeros((NP,), jnp.float32).at[slot].set(
        gate, unique_indices=True, mode="promise_in_bounds")

    gblk = jnp.arange(NB, dtype=jnp.int32)
    block_expert = jnp.sum(
        (gblk[:, None] >= blk_start[None, :]).astype(jnp.int32), axis=1) - 1
    valid = (gblk < total_blocks).astype(jnp.int32)
    meta = jnp.concatenate([block_expert, valid])

    xs = _sc_scatter_dispatch(x, slot)

    grid_spec = pltpu.PrefetchScalarGridSpec(
        num_scalar_prefetch=1,
        grid=(NB,),
        in_specs=[
            pl.BlockSpec((B, D_MODEL), lambda g, m: (g, 0)),
            pl.BlockSpec((1, D_MODEL, D_MODEL), lambda g, m: (m[g], 0, 0)),
            pl.BlockSpec((1, 1, D_MODEL), lambda g, m: (m[g], 0, 0)),
            pl.BlockSpec((1, D_MODEL, D_OUT), lambda g, m: (m[g], 0, 0)),
            pl.BlockSpec((1, 1, D_OUT), lambda g, m: (m[g], 0, 0)),
            pl.BlockSpec((1, 1, B), lambda g, m: (g, 0, 0)),
        ],
        out_specs=pl.BlockSpec((B, D_OUT), lambda g, m: (g, 0)),
    )
    y = pl.pallas_call(
        _gemm_block,
        grid_spec=grid_spec,
        out_shape=jax.ShapeDtypeStruct((NP, D_OUT), jnp.float32),
    )(meta, xs, W1, b1.reshape(N_EXPERTS, 1, D_MODEL), W2,
      b2.reshape(N_EXPERTS, 1, D_OUT), gate_sorted.reshape(NB, 1, B))

    return _sc_gather_combine(y, slot)
